# baseline (speedup 1.0000x reference)
import functools

import jax
import jax.numpy as jnp
from jax import lax
from jax.experimental import pallas as pl
from jax.experimental.pallas import tpu as pltpu
from jax.experimental.pallas import tpu_sc as plsc

VOCAB = 1000000
DIM = 32
B = 4096
L = 50
NC = 2
NS = 16
NW = NC * NS
CH = B // NW
STEPS = L

TBLK = 4096

_mesh = plsc.VectorSubcoreMesh(core_axis_name="c", subcore_axis_name="s")


def _tr_body(st, it, jt, kt, so, io, jo, ko):
    for src, dst in ((st, so), (it, io), (jt, jo), (kt, ko)):
        for j in range(4):
            dst[:, DIM * j:DIM * (j + 1)] = jnp.transpose(
                src[:, 1024 * j:1024 * (j + 1)])


_transpose_tables = pl.pallas_call(
    _tr_body,
    grid=(pl.cdiv(VOCAB, TBLK),),
    in_specs=[pl.BlockSpec((DIM, TBLK), lambda i: (0, i))] * 4,
    out_specs=[pl.BlockSpec((TBLK // 4, 128), lambda i: (i, 0))] * 4,
    out_shape=[jax.ShapeDtypeStruct(
        (pl.cdiv(VOCAB, TBLK) * (TBLK // 4), 128), jnp.float32)] * 4,
)


@functools.partial(
    pl.kernel,
    out_type=jax.ShapeDtypeStruct((L, DIM, 4 * B), jnp.float32),
    mesh=_mesh,
    scratch_types=[
        pltpu.VMEM((CH,), jnp.int32),
        [[pltpu.VMEM((CH // 4, 128), jnp.float32) for _ in range(4)]
         for _ in range(2)],
        pltpu.VMEM((DIM, 4, CH), jnp.float32),
        [pltpu.SemaphoreType.DMA for _ in range(2)],
        pltpu.SemaphoreType.DMA,
    ],
    compiler_params=pltpu.CompilerParams(needs_layout_passes=False),
)
def _emb(x_hbm, s_hbm, vi_hbm, vj_hbm, vk_hbm, out_hbm,
         idxb, rbufs, obuf, gsems, osem):
    wid = lax.axis_index("s") * NC + lax.axis_index("c")
    woff = pl.multiple_of(wid * CH, CH)
    lanes = lax.iota(jnp.int32, 16)
    tables = (s_hbm, vi_hbm, vj_hbm, vk_hbm)
    dvecs = [16 * h + lanes for h in range(2)]
    qvecs = [jnp.full((16,), q, jnp.int32) for q in range(4)]

    HC = CH // 4

    def fire_sub(sub, bset):
        def grp(g, carry):
            vec = idxb[pl.ds(sub * HC + g * 16, 16)]
            for l in range(16):
                v = vec[l]
                v4 = ((v >> 12) << 10) | (v & 1023)
                for q in range(4):
                    pltpu.make_async_copy(
                        tables[q].at[pl.ds(v4, 1)],
                        rbufs[bset][q].at[pl.ds(g * 16 + l, 1)],
                        gsems[bset]).start()
            return carry

        lax.fori_loop(0, HC // 16, grp, 0)

    def gwait(bset):
        for q in range(4):
            pltpu.make_async_copy(
                tables[q].at[pl.ds(0, HC)], rbufs[bset][q],
                gsems[bset]).wait()

    def interleave(sub, bset):
        rbuf = rbufs[bset]

        def grp(g, carry2):
            vec = idxb[pl.ds(sub * HC + g * 16, 16)]
            for l in range(16):
                r = g * 16 + l
                off = ((vec[l] >> 10) & 3) * DIM
                s_vec = jnp.full((16,), sub * HC + r, jnp.int32)
                for q in range(4):
                    for h in range(2):
                        v = rbuf[q][r, pl.ds(off + 16 * h, 16)]
                        plsc.store_scatter(
                            obuf, [dvecs[h], qvecs[q], s_vec], v)
            return carry2

        lax.fori_loop(0, HC // 16, grp, 0)

    def owait():
        for q in range(4):
            pltpu.make_async_copy(
                obuf.at[:, q], out_hbm.at[0, :, pl.ds(0, CH)],
                osem).wait()

    pltpu.sync_copy(x_hbm.at[0, pl.ds(woff, CH)], idxb)
    fire_sub(0, 0)

    def l_body(t, carry):
        gwait(0)

        @pl.when(t > 0)
        def _():
            owait()

        fire_sub(1, 1)
        interleave(0, 0)

        gwait(1)
        fire_sub(2, 0)
        interleave(1, 1)

        gwait(0)
        fire_sub(3, 1)
        interleave(2, 0)

        gwait(1)
        interleave(3, 1)

        @pl.when(t + 1 < STEPS)
        def _():
            pltpu.sync_copy(x_hbm.at[t + 1, pl.ds(woff, CH)], idxb)
            fire_sub(0, 0)

        for q in range(4):
            pltpu.make_async_copy(
                obuf.at[:, q],
                out_hbm.at[t, :, pl.ds(q * B + woff, CH)],
                osem).start()
        return carry

    lax.fori_loop(0, STEPS, l_body, 0)
    owait()


def kernel(x, scalar, vector_i, vector_j, vector_k):
    xt = jnp.transpose(x).astype(jnp.int32)
    tabs = _transpose_tables(
        jnp.transpose(scalar), jnp.transpose(vector_i),
        jnp.transpose(vector_j), jnp.transpose(vector_k))
    out = _emb(xt, *tabs)
    out = out.reshape(L, DIM, 4, B)
    return jnp.transpose(out, (3, 0, 1, 2))


# speedup vs baseline: 1.1461x; 1.1461x over previous
import functools

import jax
import jax.numpy as jnp
from jax import lax
from jax.experimental import pallas as pl
from jax.experimental.pallas import tpu as pltpu
from jax.experimental.pallas import tpu_sc as plsc

VOCAB = 1000000
DIM = 32
B = 4096
L = 50
NC = 2
NS = 16
NW = NC * NS
CH = B // NW
STEPS = L

TBLK = 4096

_mesh = plsc.VectorSubcoreMesh(core_axis_name="c", subcore_axis_name="s")


def _tr_body(st, it, jt, kt, so, io, jo, ko):
    so[...] = jnp.transpose(st[...])
    io[...] = jnp.transpose(it[...])
    jo[...] = jnp.transpose(jt[...])
    ko[...] = jnp.transpose(kt[...])


_transpose_tables = pl.pallas_call(
    _tr_body,
    grid=(pl.cdiv(VOCAB, TBLK),),
    in_specs=[pl.BlockSpec((DIM, TBLK), lambda i: (0, i))] * 4,
    out_specs=[pl.BlockSpec((TBLK, DIM), lambda i: (i, 0))] * 4,
    out_shape=[jax.ShapeDtypeStruct((VOCAB, DIM), jnp.float32)] * 4,
)


@functools.partial(
    pl.kernel,
    out_type=jax.ShapeDtypeStruct((L, DIM, 4 * B), jnp.float32),
    mesh=_mesh,
    scratch_types=[
        pltpu.VMEM((CH,), jnp.int32),
        [[pltpu.VMEM((CH // 2, DIM), jnp.float32) for _ in range(4)]
         for _ in range(2)],
        pltpu.VMEM((DIM, 4, CH), jnp.float32),
        [pltpu.SemaphoreType.DMA for _ in range(2)],
        pltpu.SemaphoreType.DMA,
    ],
    compiler_params=pltpu.CompilerParams(needs_layout_passes=False),
)
def _emb(x_hbm, s_hbm, vi_hbm, vj_hbm, vk_hbm, out_hbm,
         idxb, rbufs, obuf, gsems, osem):
    wid = lax.axis_index("s") * NC + lax.axis_index("c")
    woff = pl.multiple_of(wid * CH, CH)
    lanes = lax.iota(jnp.int32, 16)
    tables = (s_hbm, vi_hbm, vj_hbm, vk_hbm)
    dvecs = [16 * h + lanes for h in range(2)]
    qvecs = [jnp.full((16,), q, jnp.int32) for q in range(4)]

    HC = CH // 2

    def fire_half(half, bset):
        def grp(g, carry):
            vec = idxb[pl.ds(half * HC + g * 16, 16)]
            for l in range(16):
                v = vec[l]
                for q in range(4):
                    pltpu.make_async_copy(
                        tables[q].at[pl.ds(v, 1)],
                        rbufs[bset][q].at[pl.ds(g * 16 + l, 1)],
                        gsems[bset]).start()
            return carry

        lax.fori_loop(0, HC // 16, grp, 0)

    def gwait(bset):
        for q in range(4):
            pltpu.make_async_copy(
                tables[q].at[pl.ds(0, HC)], rbufs[bset][q],
                gsems[bset]).wait()

    def interleave(half, bset):
        rbuf = rbufs[bset]

        def row(r, carry2):
            s_vec = jnp.full((16,), half * HC + r, jnp.int32)
            for q in range(4):
                for h in range(2):
                    v = rbuf[q][r, pl.ds(16 * h, 16)]
                    plsc.store_scatter(
                        obuf, [dvecs[h], qvecs[q], s_vec], v)
            return carry2

        lax.fori_loop(0, HC, row, 0, unroll=4)

    def owait():
        for q in range(4):
            pltpu.make_async_copy(
                obuf.at[:, q], out_hbm.at[0, :, pl.ds(0, CH)],
                osem).wait()

    pltpu.sync_copy(x_hbm.at[0, pl.ds(woff, CH)], idxb)
    fire_half(0, 0)

    def l_body(t, carry):
        gwait(0)

        @pl.when(t > 0)
        def _():
            owait()

        fire_half(1, 1)
        interleave(0, 0)

        gwait(1)

        @pl.when(t + 1 < STEPS)
        def _():
            pltpu.sync_copy(x_hbm.at[t + 1, pl.ds(woff, CH)], idxb)
            fire_half(0, 0)

        interleave(1, 1)

        for q in range(4):
            pltpu.make_async_copy(
                obuf.at[:, q],
                out_hbm.at[t, :, pl.ds(q * B + woff, CH)],
                osem).start()
        return carry

    lax.fori_loop(0, STEPS, l_body, 0)
    owait()


def kernel(x, scalar, vector_i, vector_j, vector_k):
    xt = jnp.transpose(x).astype(jnp.int32)
    tabs = _transpose_tables(
        jnp.transpose(scalar), jnp.transpose(vector_i),
        jnp.transpose(vector_j), jnp.transpose(vector_k))
    out = _emb(xt, *tabs)
    out = out.reshape(L, DIM, 4, B)
    return jnp.transpose(out, (3, 0, 1, 2))
